# Initial kernel scaffold; baseline (speedup 1.0000x reference)
import jax
import jax.numpy as jnp
from jax.experimental import pallas as pl


def kernel(attn_w, k, v, W_o, keep_idx):
    raise NotImplementedError("write your pallas kernel here")


# speedup vs baseline: 3.3496x; 3.3496x over previous
import functools

import jax
import jax.numpy as jnp
from jax import lax
from jax.experimental import pallas as pl
from jax.experimental.pallas import tpu as pltpu
from jax.experimental.pallas import tpu_sc as plsc

B, H, H_KV, Q, S, S_KEEP, D, D_MODEL = 8, 32, 8, 4, 4096, 3072, 128, 4096
GROUPS = H // H_KV
ROWS = B * H * Q

NC, NSUB, L = 2, 16, 16
NW = NC * NSUB
ROWS_PER_W = ROWS // NW
CHUNKS = S_KEEP // L
ZCHUNKS = S // L


@functools.partial(
    pl.kernel,
    out_type=(
        jax.ShapeDtypeStruct((ROWS, S_KEEP), jnp.float32),
        jax.ShapeDtypeStruct((ROWS, S), jnp.float32),
        jax.ShapeDtypeStruct((ROWS, L), jnp.float32),
    ),
    mesh=plsc.VectorSubcoreMesh(
        core_axis_name="c", subcore_axis_name="s",
        num_cores=NC, num_subcores=NSUB),
    compiler_params=pltpu.CompilerParams(needs_layout_passes=False),
    scratch_types=[
        pltpu.VMEM((S_KEEP,), jnp.int32),
        pltpu.VMEM((S,), jnp.float32),
        pltpu.VMEM((S,), jnp.float32),
        pltpu.VMEM((S_KEEP,), jnp.float32),
        pltpu.VMEM((S_KEEP,), jnp.float32),
        pltpu.VMEM((S,), jnp.float32),
        pltpu.VMEM((S,), jnp.float32),
        pltpu.VMEM((ROWS_PER_W, L), jnp.float32),
        pltpu.SemaphoreType.DMA((2,)),
        pltpu.SemaphoreType.DMA((2,)),
        pltpu.SemaphoreType.DMA((2,)),
    ],
)
def _sc_prune(attn_hbm, idx_hbm, aw_hbm, ws_hbm, den_hbm,
              idx_v, row_v0, row_v1, aw_v0, aw_v1, ws_v0, ws_v1, den_v,
              in_sem, aw_sem, ws_sem):
    row_v = [row_v0, row_v1]
    aw_v = [aw_v0, aw_v1]
    ws_v = [ws_v0, ws_v1]
    wid = lax.axis_index("s") * NC + lax.axis_index("c")
    base = wid * ROWS_PER_W
    pltpu.sync_copy(idx_hbm, idx_v)

    zero16 = jnp.zeros((L,), jnp.float32)
    for p in (0, 1):
        @plsc.parallel_loop(0, ZCHUNKS, unroll=8)
        def _clear(i, _p=p):
            ws_v[_p][pl.ds(i * L, L)] = zero16

    in_d = [None, None]
    aw_d = [None, None]
    ws_d = [None, None]
    in_d[0] = pltpu.async_copy(attn_hbm.at[base], row_v[0], in_sem.at[0])

    for rr in range(ROWS_PER_W):
        p = rr & 1
        if rr + 1 < ROWS_PER_W:
            in_d[1 - p] = pltpu.async_copy(
                attn_hbm.at[base + rr + 1], row_v[1 - p], in_sem.at[1 - p])
        in_d[p].wait()
        if rr >= 2:
            aw_d[p].wait()
            ws_d[p].wait()

            @plsc.parallel_loop(0, CHUNKS, unroll=8)
            def _rezero(j, _p=p):
                idx16 = idx_v[pl.ds(j * L, L)]
                plsc.store_scatter(ws_v[_p], [idx16], zero16)

        @plsc.parallel_loop(0, CHUNKS, unroll=8,
                            carry=jnp.zeros((L,), jnp.float32))
        def _fused(j, acc, _p=p):
            idx16 = idx_v[pl.ds(j * L, L)]
            vals = plsc.load_gather(row_v[_p], [idx16])
            aw_v[_p][pl.ds(j * L, L)] = vals
            plsc.addupdate_scatter(ws_v[_p], [idx16], vals)
            return acc + vals

        den_v[rr] = _fused
        aw_d[p] = pltpu.async_copy(
            aw_v[p], aw_hbm.at[base + rr], aw_sem.at[p])
        ws_d[p] = pltpu.async_copy(
            ws_v[p], ws_hbm.at[base + rr], ws_sem.at[p])

    for p in (0, 1):
        aw_d[p].wait()
        ws_d[p].wait()
    pltpu.sync_copy(den_v, den_hbm.at[pl.ds(base, ROWS_PER_W)])


def _tc_body(ws_ref, v_ref, wo_ref, den_ref, awin_ref, out_ref, awout_ref):
    g = pl.program_id(0)
    inv = 1.0 / (jnp.sum(den_ref[:, 0], axis=-1) + 1e-6)

    c2s = []
    for b in range(B):
        ib = inv[b][:, None]
        awout_ref[b, 0] = awin_ref[b, 0] * ib
        part = lax.dot_general(
            ws_ref[b, 0], v_ref[b, 0], (((1,), (0,)), ((), ())),
            preferred_element_type=jnp.float32) * ib
        c2s.append(jnp.concatenate(
            [part[0:4], part[4:8], part[8:12], part[12:16]], axis=1))
    c2 = jnp.concatenate(c2s, axis=0)
    og = lax.dot_general(
        c2, wo_ref[...], (((1,), (1,)), ((), ())),
        preferred_element_type=jnp.float32)

    for b in range(B):
        blk = og[b * Q:(b + 1) * Q]

        @pl.when(g == 0)
        def _(b=b, blk=blk):
            out_ref[b] = blk

        @pl.when(g > 0)
        def _(b=b, blk=blk):
            out_ref[b] += blk


_tc_call = pl.pallas_call(
    _tc_body,
    grid=(H_KV,),
    in_specs=[
        pl.BlockSpec((B, 1, GROUPS * Q, S), lambda g: (0, g, 0, 0)),
        pl.BlockSpec((B, 1, S, D), lambda g: (0, g, 0, 0)),
        pl.BlockSpec((D_MODEL, GROUPS * D), lambda g: (0, g)),
        pl.BlockSpec((B, 1, GROUPS * Q, L), lambda g: (0, g, 0, 0)),
        pl.BlockSpec((B, 1, GROUPS * Q, S_KEEP), lambda g: (0, g, 0, 0)),
    ],
    out_specs=(
        pl.BlockSpec((B, Q, D_MODEL), lambda g: (0, 0, 0)),
        pl.BlockSpec((B, 1, GROUPS * Q, S_KEEP), lambda g: (0, g, 0, 0)),
    ),
    out_shape=(
        jax.ShapeDtypeStruct((B, Q, D_MODEL), jnp.float32),
        jax.ShapeDtypeStruct((B, H_KV, GROUPS * Q, S_KEEP), jnp.float32),
    ),
    compiler_params=pltpu.CompilerParams(
        vmem_limit_bytes=128 * 1024 * 1024),
)


def kernel(attn_w, k, v, W_o, keep_idx):
    del k
    attn_flat = attn_w.reshape(ROWS, S)
    aw_raw, ws, den = _sc_prune(attn_flat, keep_idx.astype(jnp.int32))
    ws4 = ws.reshape(B, H_KV, GROUPS * Q, S)
    aw4 = aw_raw.reshape(B, H_KV, GROUPS * Q, S_KEEP)
    den4 = den.reshape(B, H_KV, GROUPS * Q, L)
    out, awn = _tc_call(ws4, v, W_o, den4, aw4)
    return out, awn.reshape(B, H, Q, S_KEEP)


# speedup vs baseline: 3.3608x; 1.0033x over previous
import functools

import jax
import jax.numpy as jnp
from jax import lax
from jax.experimental import pallas as pl
from jax.experimental.pallas import tpu as pltpu
from jax.experimental.pallas import tpu_sc as plsc

B, H, H_KV, Q, S, S_KEEP, D, D_MODEL = 8, 32, 8, 4, 4096, 3072, 128, 4096
GROUPS = H // H_KV
ROWS = B * H * Q

NC, NSUB, L = 2, 16, 16
NW = NC * NSUB
ROWS_PER_W = ROWS // NW
CHUNKS = S_KEEP // L
ZCHUNKS = S // L


@functools.partial(
    pl.kernel,
    out_type=(
        jax.ShapeDtypeStruct((ROWS, S_KEEP), jnp.float32),
        jax.ShapeDtypeStruct((ROWS, S), jnp.float32),
        jax.ShapeDtypeStruct((ROWS, L), jnp.float32),
    ),
    mesh=plsc.VectorSubcoreMesh(
        core_axis_name="c", subcore_axis_name="s",
        num_cores=NC, num_subcores=NSUB),
    compiler_params=pltpu.CompilerParams(needs_layout_passes=False),
    scratch_types=[
        pltpu.VMEM((S_KEEP,), jnp.int32),
        pltpu.VMEM((S,), jnp.float32),
        pltpu.VMEM((S,), jnp.float32),
        pltpu.VMEM((S_KEEP,), jnp.float32),
        pltpu.VMEM((S_KEEP,), jnp.float32),
        pltpu.VMEM((S,), jnp.float32),
        pltpu.VMEM((S,), jnp.float32),
        pltpu.VMEM((ROWS_PER_W, L), jnp.float32),
        pltpu.SemaphoreType.DMA((2,)),
        pltpu.SemaphoreType.DMA((2,)),
        pltpu.SemaphoreType.DMA((2,)),
    ],
)
def _sc_prune(attn_hbm, idx_hbm, aw_hbm, ws_hbm, den_hbm,
              idx_v, row_v0, row_v1, aw_v0, aw_v1, ws_v0, ws_v1, den_v,
              in_sem, aw_sem, ws_sem):
    row_v = [row_v0, row_v1]
    aw_v = [aw_v0, aw_v1]
    ws_v = [ws_v0, ws_v1]
    wid = lax.axis_index("s") * NC + lax.axis_index("c")
    base = wid * ROWS_PER_W
    pltpu.sync_copy(idx_hbm, idx_v)

    zero16 = jnp.zeros((L,), jnp.float32)
    for p in (0, 1):
        @plsc.parallel_loop(0, ZCHUNKS, unroll=8)
        def _clear(i, _p=p):
            ws_v[_p][pl.ds(i * L, L)] = zero16

    in_d = [None, None]
    aw_d = [None, None]
    ws_d = [None, None]
    in_d[0] = pltpu.async_copy(attn_hbm.at[base], row_v[0], in_sem.at[0])

    for rr in range(ROWS_PER_W):
        p = rr & 1
        if rr + 1 < ROWS_PER_W:
            in_d[1 - p] = pltpu.async_copy(
                attn_hbm.at[base + rr + 1], row_v[1 - p], in_sem.at[1 - p])
        in_d[p].wait()
        if rr >= 2:
            aw_d[p].wait()
            ws_d[p].wait()

            @plsc.parallel_loop(0, CHUNKS, unroll=8)
            def _rezero(j, _p=p):
                idx16 = idx_v[pl.ds(j * L, L)]
                plsc.store_scatter(ws_v[_p], [idx16], zero16)

        @plsc.parallel_loop(0, CHUNKS, unroll=8,
                            carry=jnp.zeros((L,), jnp.float32))
        def _fused(j, acc, _p=p):
            idx16 = idx_v[pl.ds(j * L, L)]
            vals = plsc.load_gather(row_v[_p], [idx16])
            aw_v[_p][pl.ds(j * L, L)] = vals
            plsc.addupdate_scatter(ws_v[_p], [idx16], vals)
            return acc + vals

        den_v[rr] = _fused
        aw_d[p] = pltpu.async_copy(
            aw_v[p], aw_hbm.at[base + rr], aw_sem.at[p])
        ws_d[p] = pltpu.async_copy(
            ws_v[p], ws_hbm.at[base + rr], ws_sem.at[p])

    for p in (0, 1):
        aw_d[p].wait()
        ws_d[p].wait()
    pltpu.sync_copy(den_v, den_hbm.at[pl.ds(base, ROWS_PER_W)])


@functools.partial(
    pl.kernel,
    out_type=jax.ShapeDtypeStruct((ROWS, S_KEEP), jnp.float32),
    mesh=plsc.VectorSubcoreMesh(
        core_axis_name="c", subcore_axis_name="s",
        num_cores=NC, num_subcores=NSUB),
    compiler_params=pltpu.CompilerParams(needs_layout_passes=False),
    scratch_types=[
        pltpu.VMEM((ROWS_PER_W, L), jnp.float32),
        pltpu.VMEM((S_KEEP,), jnp.float32),
        pltpu.VMEM((S_KEEP,), jnp.float32),
        pltpu.SemaphoreType.DMA((2,)),
        pltpu.SemaphoreType.DMA((2,)),
    ],
)
def _sc_norm(aw_hbm, den_hbm, out_hbm, den_v, row_v0, row_v1, in_sem, out_sem):
    row_v = [row_v0, row_v1]
    wid = lax.axis_index("s") * NC + lax.axis_index("c")
    base = wid * ROWS_PER_W
    pltpu.sync_copy(den_hbm.at[pl.ds(base, ROWS_PER_W)], den_v)

    in_d = [None, None]
    out_d = [None, None]
    in_d[0] = pltpu.async_copy(aw_hbm.at[base], row_v[0], in_sem.at[0])
    for rr in range(ROWS_PER_W):
        p = rr & 1
        if rr + 1 < ROWS_PER_W:
            in_d[1 - p] = pltpu.async_copy(
                aw_hbm.at[base + rr + 1], row_v[1 - p], in_sem.at[1 - p])
        in_d[p].wait()
        if rr >= 2:
            out_d[p].wait()
        total = jnp.sum(den_v[rr])
        inv16 = 1.0 / (jnp.full((L,), total, jnp.float32) + 1e-6)

        @plsc.parallel_loop(0, CHUNKS, unroll=8)
        def _scale(j, _p=p, _inv=inv16):
            sl = pl.ds(j * L, L)
            row_v[_p][sl] = row_v[_p][sl] * _inv

        out_d[p] = pltpu.async_copy(
            row_v[p], out_hbm.at[base + rr], out_sem.at[p])
    for p in (0, 1):
        out_d[p].wait()


def _tc_body(ws_ref, v_ref, wo_ref, den_ref, out_ref):
    g = pl.program_id(0)
    inv = 1.0 / (jnp.sum(den_ref[:, 0], axis=-1) + 1e-6)

    c2s = []
    for b in range(B):
        ib = inv[b][:, None]
        part = lax.dot_general(
            ws_ref[b, 0], v_ref[b, 0], (((1,), (0,)), ((), ())),
            preferred_element_type=jnp.float32) * ib
        c2s.append(jnp.concatenate(
            [part[0:4], part[4:8], part[8:12], part[12:16]], axis=1))
    c2 = jnp.concatenate(c2s, axis=0)
    og = lax.dot_general(
        c2, wo_ref[...], (((1,), (1,)), ((), ())),
        preferred_element_type=jnp.float32)

    for b in range(B):
        blk = og[b * Q:(b + 1) * Q]

        @pl.when(g == 0)
        def _(b=b, blk=blk):
            out_ref[b] = blk

        @pl.when(g > 0)
        def _(b=b, blk=blk):
            out_ref[b] += blk


_tc_call = pl.pallas_call(
    _tc_body,
    grid=(H_KV,),
    in_specs=[
        pl.BlockSpec((B, 1, GROUPS * Q, S), lambda g: (0, g, 0, 0)),
        pl.BlockSpec((B, 1, S, D), lambda g: (0, g, 0, 0)),
        pl.BlockSpec((D_MODEL, GROUPS * D), lambda g: (0, g)),
        pl.BlockSpec((B, 1, GROUPS * Q, L), lambda g: (0, g, 0, 0)),
    ],
    out_specs=pl.BlockSpec((B, Q, D_MODEL), lambda g: (0, 0, 0)),
    out_shape=jax.ShapeDtypeStruct((B, Q, D_MODEL), jnp.float32),
    compiler_params=pltpu.CompilerParams(
        vmem_limit_bytes=128 * 1024 * 1024),
)


def kernel(attn_w, k, v, W_o, keep_idx):
    del k
    attn_flat = attn_w.reshape(ROWS, S)
    aw_raw, ws, den = _sc_prune(attn_flat, keep_idx.astype(jnp.int32))
    ws4 = ws.reshape(B, H_KV, GROUPS * Q, S)
    den4 = den.reshape(B, H_KV, GROUPS * Q, L)
    awn = _sc_norm(aw_raw, den)
    out = _tc_call(ws4, v, W_o, den4)
    return out, awn.reshape(B, H, Q, S_KEEP)


# speedup vs baseline: 4.0742x; 1.2123x over previous
import functools

import jax
import jax.numpy as jnp
from jax import lax
from jax.experimental import pallas as pl
from jax.experimental.pallas import tpu as pltpu
from jax.experimental.pallas import tpu_sc as plsc

B, H, H_KV, Q, S, S_KEEP, D, D_MODEL = 8, 32, 8, 4, 4096, 3072, 128, 4096
GROUPS = H // H_KV
ROWS = B * H * Q

NC, NSUB, L = 2, 16, 16
NW = NC * NSUB
ROWS_PER_W = ROWS // NW
CHUNKS = S_KEEP // L
ZCHUNKS = S // L


@functools.partial(
    pl.kernel,
    out_type=(
        jax.ShapeDtypeStruct((B, H, Q, S_KEEP), jnp.float32),
        jax.ShapeDtypeStruct((ROWS, S), jnp.float32),
        jax.ShapeDtypeStruct((ROWS, L), jnp.float32),
    ),
    mesh=plsc.VectorSubcoreMesh(
        core_axis_name="c", subcore_axis_name="s",
        num_cores=NC, num_subcores=NSUB),
    compiler_params=pltpu.CompilerParams(needs_layout_passes=False),
    scratch_types=[
        pltpu.VMEM((S_KEEP,), jnp.int32),
        pltpu.VMEM((S,), jnp.float32),
        pltpu.VMEM((S,), jnp.float32),
        pltpu.VMEM((S_KEEP,), jnp.float32),
        pltpu.VMEM((S_KEEP,), jnp.float32),
        pltpu.VMEM((S,), jnp.float32),
        pltpu.VMEM((S,), jnp.float32),
        pltpu.VMEM((ROWS_PER_W, L), jnp.float32),
        pltpu.SemaphoreType.DMA((2,)),
        pltpu.SemaphoreType.DMA((2,)),
        pltpu.SemaphoreType.DMA((2,)),
    ],
)
def _sc_prune(attn_hbm, idx_hbm, aw_hbm, ws_hbm, den_hbm,
              idx_v, row_v0, row_v1, aw_v0, aw_v1, ws_v0, ws_v1, den_v,
              in_sem, aw_sem, ws_sem):
    row_v = [row_v0, row_v1]
    aw_v = [aw_v0, aw_v1]
    ws_v = [ws_v0, ws_v1]
    wid = lax.axis_index("s") * NC + lax.axis_index("c")
    base = wid * ROWS_PER_W
    b_t = wid // (H * Q // ROWS_PER_W)
    h0 = (wid % 4) * (ROWS_PER_W // Q)
    pltpu.sync_copy(idx_hbm, idx_v)

    zero16 = jnp.zeros((L,), jnp.float32)
    for p in (0, 1):
        @plsc.parallel_loop(0, ZCHUNKS, unroll=8)
        def _clear(i, _p=p):
            ws_v[_p][pl.ds(i * L, L)] = zero16

    in_d = [None, None]
    aw_d = [None, None]
    ws_d = [None, None]
    in_d[0] = pltpu.async_copy(
        attn_hbm.at[b_t, h0, 0], row_v[0], in_sem.at[0])

    for rr in range(ROWS_PER_W):
        p = rr & 1
        if rr + 1 < ROWS_PER_W:
            in_d[1 - p] = pltpu.async_copy(
                attn_hbm.at[b_t, h0 + (rr + 1) // Q, (rr + 1) % Q],
                row_v[1 - p], in_sem.at[1 - p])
        in_d[p].wait()
        if rr >= 2:
            aw_d[p].wait()
            ws_d[p].wait()

            @plsc.parallel_loop(0, CHUNKS, unroll=8)
            def _rezero(j, _p=p):
                idx16 = idx_v[pl.ds(j * L, L)]
                plsc.store_scatter(ws_v[_p], [idx16], zero16)

        @plsc.parallel_loop(0, CHUNKS, unroll=8,
                            carry=jnp.zeros((L,), jnp.float32))
        def _fused(j, acc, _p=p):
            idx16 = idx_v[pl.ds(j * L, L)]
            vals = plsc.load_gather(row_v[_p], [idx16])
            aw_v[_p][pl.ds(j * L, L)] = vals
            plsc.addupdate_scatter(ws_v[_p], [idx16], vals)
            return acc + vals

        den_v[rr] = _fused
        aw_d[p] = pltpu.async_copy(
            aw_v[p], aw_hbm.at[b_t, h0 + rr // Q, rr % Q], aw_sem.at[p])
        ws_d[p] = pltpu.async_copy(
            ws_v[p], ws_hbm.at[base + rr], ws_sem.at[p])

    for p in (0, 1):
        aw_d[p].wait()
        ws_d[p].wait()
    pltpu.sync_copy(den_v, den_hbm.at[pl.ds(base, ROWS_PER_W)])


@functools.partial(
    pl.kernel,
    out_type=jax.ShapeDtypeStruct((B, H, Q, S_KEEP), jnp.float32),
    mesh=plsc.VectorSubcoreMesh(
        core_axis_name="c", subcore_axis_name="s",
        num_cores=NC, num_subcores=NSUB),
    compiler_params=pltpu.CompilerParams(needs_layout_passes=False),
    scratch_types=[
        pltpu.VMEM((ROWS_PER_W, L), jnp.float32),
        pltpu.VMEM((S_KEEP,), jnp.float32),
        pltpu.VMEM((S_KEEP,), jnp.float32),
        pltpu.SemaphoreType.DMA((2,)),
        pltpu.SemaphoreType.DMA((2,)),
    ],
)
def _sc_norm(aw_hbm, den_hbm, out_hbm, den_v, row_v0, row_v1, in_sem, out_sem):
    row_v = [row_v0, row_v1]
    wid = lax.axis_index("s") * NC + lax.axis_index("c")
    base = wid * ROWS_PER_W
    b_t = wid // (H * Q // ROWS_PER_W)
    h0 = (wid % 4) * (ROWS_PER_W // Q)
    pltpu.sync_copy(den_hbm.at[pl.ds(base, ROWS_PER_W)], den_v)

    in_d = [None, None]
    out_d = [None, None]
    in_d[0] = pltpu.async_copy(aw_hbm.at[b_t, h0, 0], row_v[0], in_sem.at[0])
    for rr in range(ROWS_PER_W):
        p = rr & 1
        if rr + 1 < ROWS_PER_W:
            in_d[1 - p] = pltpu.async_copy(
                aw_hbm.at[b_t, h0 + (rr + 1) // Q, (rr + 1) % Q],
                row_v[1 - p], in_sem.at[1 - p])
        in_d[p].wait()
        if rr >= 2:
            out_d[p].wait()
        total = jnp.sum(den_v[rr])
        inv16 = 1.0 / (jnp.full((L,), total, jnp.float32) + 1e-6)

        @plsc.parallel_loop(0, CHUNKS, unroll=8)
        def _scale(j, _p=p, _inv=inv16):
            sl = pl.ds(j * L, L)
            row_v[_p][sl] = row_v[_p][sl] * _inv

        out_d[p] = pltpu.async_copy(
            row_v[p], out_hbm.at[b_t, h0 + rr // Q, rr % Q], out_sem.at[p])
    for p in (0, 1):
        out_d[p].wait()


def _tc_body(ws_ref, v_ref, wo_ref, den_ref, out_ref):
    g = pl.program_id(0)
    inv = 1.0 / (jnp.sum(den_ref[:, 0], axis=-1) + 1e-6)

    c2s = []
    for b in range(B):
        ib = inv[b][:, None]
        part = lax.dot_general(
            ws_ref[b, 0], v_ref[b, 0], (((1,), (0,)), ((), ())),
            preferred_element_type=jnp.float32) * ib
        c2s.append(jnp.concatenate(
            [part[0:4], part[4:8], part[8:12], part[12:16]], axis=1))
    c2 = jnp.concatenate(c2s, axis=0)
    og = lax.dot_general(
        c2, wo_ref[...], (((1,), (1,)), ((), ())),
        preferred_element_type=jnp.float32)

    for b in range(B):
        blk = og[b * Q:(b + 1) * Q]

        @pl.when(g == 0)
        def _(b=b, blk=blk):
            out_ref[b] = blk

        @pl.when(g > 0)
        def _(b=b, blk=blk):
            out_ref[b] += blk


_tc_call = pl.pallas_call(
    _tc_body,
    grid=(H_KV,),
    in_specs=[
        pl.BlockSpec((B, 1, GROUPS * Q, S), lambda g: (0, g, 0, 0)),
        pl.BlockSpec((B, 1, S, D), lambda g: (0, g, 0, 0)),
        pl.BlockSpec((D_MODEL, GROUPS * D), lambda g: (0, g)),
        pl.BlockSpec((B, 1, GROUPS * Q, L), lambda g: (0, g, 0, 0)),
    ],
    out_specs=pl.BlockSpec((B, Q, D_MODEL), lambda g: (0, 0, 0)),
    out_shape=jax.ShapeDtypeStruct((B, Q, D_MODEL), jnp.float32),
    compiler_params=pltpu.CompilerParams(
        vmem_limit_bytes=128 * 1024 * 1024),
)


def kernel(attn_w, k, v, W_o, keep_idx):
    del k
    aw_raw, ws, den = _sc_prune(attn_w, keep_idx.astype(jnp.int32))
    ws4 = ws.reshape(B, H_KV, GROUPS * Q, S)
    den4 = den.reshape(B, H_KV, GROUPS * Q, L)
    awn = _sc_norm(aw_raw, den)
    out = _tc_call(ws4, v, W_o, den4)
    return out, awn
